# baseline (speedup 1.0000x reference)
import functools

import jax
import jax.numpy as jnp
from jax import lax
from jax.experimental import pallas as pl
from jax.experimental.pallas import tpu as pltpu
from jax.experimental.pallas import tpu_sc as plsc

_DIM = 64
_NUM_WORKERS = 32
_CHUNK = 128


def _build(num_rows: int):
    rows_per_worker = num_rows // _NUM_WORKERS
    n_chunks = rows_per_worker // _CHUNK
    out_lines = num_rows * _DIM // 128
    lines_per_chunk = _CHUNK * _DIM // 128
    lines_per_worker = rows_per_worker * _DIM // 128
    mesh = plsc.VectorSubcoreMesh(core_axis_name="c", subcore_axis_name="s")

    @functools.partial(
        pl.kernel,
        mesh=mesh,
        compiler_params=pltpu.CompilerParams(use_tc_tiling_on_sc=True),
        out_type=jax.ShapeDtypeStruct((out_lines, 128), jnp.float32),
        scratch_types=[
            pltpu.VMEM((rows_per_worker,), jnp.int32),
            pltpu.VMEM((rows_per_worker,), jnp.int32),
            pltpu.VMEM((_CHUNK, 128), jnp.float32),
            pltpu.VMEM((_CHUNK, 128), jnp.float32),
            pltpu.VMEM((lines_per_chunk, 128), jnp.float32),
            pltpu.VMEM((lines_per_chunk, 128), jnp.float32),
            pltpu.VMEM((rows_per_worker,), jnp.int32),
            pltpu.SemaphoreType.DMA,
            pltpu.SemaphoreType.DMA,
        ],
    )
    def gather_kernel(idx_hbm, table_hbm, out_hbm, line_v, par_v,
                      buf_a, buf_b, stage_a, stage_b, raw_v, sem_a, sem_b):
        wid = lax.axis_index("s") * 2 + lax.axis_index("c")
        base = wid * rows_per_worker
        obase = wid * lines_per_worker
        pltpu.sync_copy(idx_hbm.at[pl.ds(base, rows_per_worker)], raw_v)

        def prep(i, carry):
            v = raw_v[pl.ds(i * 16, 16)]
            line_v[pl.ds(i * 16, 16)] = lax.shift_right_logical(v, 1)
            par_v[pl.ds(i * 16, 16)] = lax.shift_left(
                lax.bitwise_and(v, 1), 6)
            return carry

        lax.fori_loop(0, rows_per_worker // 16, prep, 0)

        def start_chunk(m, buf, sem):
            pltpu.async_copy(
                table_hbm.at[line_v.at[pl.ds(m * _CHUNK, _CHUNK)]], buf, sem)

        def wait_chunk(buf, sem):
            pltpu.make_async_copy(
                out_hbm.at[pl.ds(obase, _CHUNK)], buf, sem).wait()

        def extract(m, buf, stage):
            def row_group(g, carry):
                pv = par_v[pl.ds(m * _CHUNK + g * 16, 16)]
                for u in range(16):
                    i = g * 16 + u
                    c = pl.multiple_of(pv[u], 16)
                    for k in range(4):
                        vals = buf[i, pl.ds(c + k * 16, 16)]
                        stage[g * 8 + u // 2,
                              pl.ds((u % 2) * 64 + k * 16, 16)] = vals
                return carry

            lax.fori_loop(0, _CHUNK // 16, row_group, 0)

        def out_chunk(m, stage):
            pltpu.sync_copy(
                stage, out_hbm.at[pl.ds(obase + m * lines_per_chunk,
                                        lines_per_chunk)])

        start_chunk(0, buf_a, sem_a)

        def body(t, carry):
            start_chunk(2 * t + 1, buf_b, sem_b)
            wait_chunk(buf_a, sem_a)
            extract(2 * t, buf_a, stage_a)
            out_chunk(2 * t, stage_a)
            start_chunk(2 * t + 2, buf_a, sem_a)
            wait_chunk(buf_b, sem_b)
            extract(2 * t + 1, buf_b, stage_b)
            out_chunk(2 * t + 1, stage_b)
            return carry

        lax.fori_loop(0, n_chunks // 2 - 1, body, 0)
        start_chunk(n_chunks - 1, buf_b, sem_b)
        wait_chunk(buf_a, sem_a)
        extract(n_chunks - 2, buf_a, stage_a)
        out_chunk(n_chunks - 2, stage_a)
        wait_chunk(buf_b, sem_b)
        extract(n_chunks - 1, buf_b, stage_b)
        out_chunk(n_chunks - 1, stage_b)

    return gather_kernel


def kernel(x, table):
    batch, seq = x.shape
    num_rows = batch * seq
    table2 = jnp.reshape(table, (table.shape[0] // 2, 128))
    idx = jnp.reshape(x, (num_rows,))
    out = _build(num_rows)(idx, table2)
    return out.reshape(batch, seq, _DIM)
